# baseline (speedup 1.0000x reference)
import functools

import jax
import jax.numpy as jnp
from jax import lax
from jax.experimental import pallas as pl
from jax.experimental.pallas import tpu as pltpu
from jax.experimental.pallas import tpu_sc as plsc

HIDDEN = 768
EPS = 1e-5
PAD_IDX = 0

_NUM_CORES = 2
_NUM_SUBCORES = 16

_GATHER_WINDOW = 128
_HALF = HIDDEN // 2

_LN_ROWS = 1024


def _sc_gather(table, ids):
    n = ids.shape[0]
    half_idx = jnp.stack([ids * 2, ids * 2 + 1], axis=-1).reshape(1, 2 * n)
    table_half = table.reshape(-1, _HALF)
    mesh = plsc.VectorSubcoreMesh(core_axis_name="core", subcore_axis_name="subcore")

    @functools.partial(
        pl.kernel,
        out_type=jax.ShapeDtypeStruct((2 * n, _HALF), jnp.float32),
        mesh=mesh,
    )
    def gather_kernel(table_hbm, idx_hbm, out_hbm):
        def body(idx_vmem, rows_vmem):
            pltpu.sync_copy(table_hbm.at[idx_vmem.at[0]], rows_vmem)

        pltpu.emit_pipeline(
            body,
            grid=(2 * n // _GATHER_WINDOW,),
            in_specs=[
                pl.BlockSpec((1, _GATHER_WINDOW), index_map=lambda i: (0, i))
            ],
            out_specs=[
                pl.BlockSpec((_GATHER_WINDOW, _HALF), index_map=lambda i: (i, 0))
            ],
            core_axis_name=("core", "subcore"),
            dimension_semantics=(pltpu.PARALLEL,),
        )(idx_hbm, out_hbm)

    return gather_kernel(table_half, half_idx).reshape(n, HIDDEN)


def _tc_layernorm(rows, ids_col, gamma_row, beta_row):
    n = rows.shape[0]

    def body(x_ref, ids_ref, g_ref, b_ref, o_ref):
        x = x_ref[...]
        mean = jnp.mean(x, axis=1, keepdims=True)
        xc = x - mean
        var = jnp.mean(xc * xc, axis=1, keepdims=True)
        normed = xc * lax.rsqrt(var + EPS)
        out = normed * g_ref[...] + b_ref[...]
        pad = ids_ref[...] == PAD_IDX
        o_ref[...] = jnp.where(pad, b_ref[...], out)

    return pl.pallas_call(
        body,
        grid=(n // _LN_ROWS,),
        in_specs=[
            pl.BlockSpec((_LN_ROWS, HIDDEN), lambda i: (i, 0)),
            pl.BlockSpec((_LN_ROWS, 1), lambda i: (i, 0)),
            pl.BlockSpec((1, HIDDEN), lambda i: (0, 0)),
            pl.BlockSpec((1, HIDDEN), lambda i: (0, 0)),
        ],
        out_specs=pl.BlockSpec((_LN_ROWS, HIDDEN), lambda i: (i, 0)),
        out_shape=jax.ShapeDtypeStruct((n, HIDDEN), jnp.float32),
    )(rows, ids_col, gamma_row, beta_row)


def kernel(input_ids, table, gamma, beta):
    b, s = input_ids.shape
    ids = input_ids.reshape(-1).astype(jnp.int32)
    rows = _sc_gather(table, ids)
    out = _tc_layernorm(
        rows,
        ids.reshape(-1, 1),
        gamma.reshape(1, HIDDEN),
        beta.reshape(1, HIDDEN),
    )
    return out.reshape(b, s, HIDDEN)


# speedup vs baseline: 2.7158x; 2.7158x over previous
import functools

import jax
import jax.numpy as jnp
from jax import lax
from jax.experimental import pallas as pl
from jax.experimental.pallas import tpu as pltpu
from jax.experimental.pallas import tpu_sc as plsc

HIDDEN = 768
EPS = 1e-5
PAD_IDX = 0

_NUM_CORES = 2
_NUM_SUBCORES = 16

_CHUNK = 64

_LN_ROWS = 1024


def _sc_gather(table, ids):
    n = ids.shape[0]
    n_tiles = _NUM_CORES * _NUM_SUBCORES
    rows_per_tile = n // n_tiles
    n_chunks = rows_per_tile // _CHUNK
    mesh = plsc.VectorSubcoreMesh(core_axis_name="core", subcore_axis_name="subcore")

    @functools.partial(
        pl.kernel,
        out_type=jax.ShapeDtypeStruct((n, HIDDEN), jnp.float32),
        mesh=mesh,
        scratch_types=[
            pltpu.VMEM((rows_per_tile,), jnp.int32),
            pltpu.VMEM((_CHUNK, HIDDEN), jnp.float32),
            pltpu.VMEM((_CHUNK, HIDDEN), jnp.float32),
            pltpu.SemaphoreType.DMA,
            pltpu.SemaphoreType.DMA,
            pltpu.SemaphoreType.DMA,
            pltpu.SemaphoreType.DMA,
        ],
    )
    def gather_kernel(table_hbm, idx_hbm, out_hbm,
                      idx_v, buf0, buf1, g0, g1, s0, s1):
        wid = lax.axis_index("subcore") * _NUM_CORES + lax.axis_index("core")
        base = wid * rows_per_tile
        pltpu.sync_copy(idx_hbm.at[pl.ds(base, rows_per_tile)], idx_v)

        def gather_copy(c, buf, sem):
            return pltpu.make_async_copy(
                table_hbm.at[idx_v.at[pl.ds(c * _CHUNK, _CHUNK)]], buf, sem
            )

        def store_copy(c, buf, sem):
            return pltpu.make_async_copy(
                buf, out_hbm.at[pl.ds(base + c * _CHUNK, _CHUNK)], sem
            )

        gather_copy(0, buf0, g0).start()

        @pl.loop(0, n_chunks, step=2)
        def _(c):
            @pl.when(c + 1 < n_chunks)
            def _():
                gather_copy(c + 1, buf1, g1).start()

            gather_copy(c, buf0, g0).wait()
            store_copy(c, buf0, s0).start()
            store_copy(c, buf0, s0).wait()

            @pl.when(c + 2 < n_chunks)
            def _():
                gather_copy(c + 2, buf0, g0).start()

            @pl.when(c + 1 < n_chunks)
            def _():
                gather_copy(c + 1, buf1, g1).wait()
                store_copy(c + 1, buf1, s1).start()
                store_copy(c + 1, buf1, s1).wait()

    return gather_kernel(table, ids)


def _tc_layernorm(rows, ids_col, gamma_row, beta_row):
    n = rows.shape[0]

    def body(x_ref, ids_ref, g_ref, b_ref, o_ref):
        x = x_ref[...]
        mean = jnp.mean(x, axis=1, keepdims=True)
        xc = x - mean
        var = jnp.mean(xc * xc, axis=1, keepdims=True)
        normed = xc * lax.rsqrt(var + EPS)
        out = normed * g_ref[...] + b_ref[...]
        pad = ids_ref[...] == PAD_IDX
        o_ref[...] = jnp.where(pad, b_ref[...], out)

    return pl.pallas_call(
        body,
        grid=(n // _LN_ROWS,),
        in_specs=[
            pl.BlockSpec((_LN_ROWS, HIDDEN), lambda i: (i, 0)),
            pl.BlockSpec((_LN_ROWS, 1), lambda i: (i, 0)),
            pl.BlockSpec((1, HIDDEN), lambda i: (0, 0)),
            pl.BlockSpec((1, HIDDEN), lambda i: (0, 0)),
        ],
        out_specs=pl.BlockSpec((_LN_ROWS, HIDDEN), lambda i: (i, 0)),
        out_shape=jax.ShapeDtypeStruct((n, HIDDEN), jnp.float32),
    )(rows, ids_col, gamma_row, beta_row)


def kernel(input_ids, table, gamma, beta):
    b, s = input_ids.shape
    ids = input_ids.reshape(-1).astype(jnp.int32)
    rows = _sc_gather(table, ids)
    out = _tc_layernorm(
        rows,
        ids.reshape(-1, 1),
        gamma.reshape(1, HIDDEN),
        beta.reshape(1, HIDDEN),
    )
    return out.reshape(b, s, HIDDEN)
